# baseline (speedup 1.0000x reference)
import jax
import jax.numpy as jnp
from jax import lax
from jax.experimental import pallas as pl

NUM_Q = 8
TB = 512


def _rvq_body(x_ref, hi_ref, mid_ref, lo_ref, cbt_ref, quant_ref, idx_ref):
    res = x_ref[...]
    tb, d = res.shape
    num_q, _, k = cbt_ref.shape
    quant = jnp.zeros_like(res)
    kiota = lax.broadcasted_iota(jnp.int32, (tb, k), 1)
    for i in range(num_q):
        wt = cbt_ref[i]
        scores = lax.dot_general(
            res, wt, (((1,), (0,)), ((), ())),
            preferred_element_type=jnp.float32)
        xsq = jnp.sum(res * res, axis=1, keepdims=True)
        wsq = jnp.sum(wt * wt, axis=0, keepdims=True)
        dist = (xsq + wsq) - 2.0 * scores
        m = jnp.min(dist, axis=1, keepdims=True)
        idx = jnp.min(jnp.where(dist == m, kiota, k), axis=1)
        onehot = (kiota == idx[:, None]).astype(jnp.bfloat16)
        dims = (((1,), (0,)), ((), ()))
        q = lax.dot_general(onehot, hi_ref[i], dims,
                            preferred_element_type=jnp.float32)
        q = q + lax.dot_general(onehot, mid_ref[i], dims,
                                preferred_element_type=jnp.float32)
        q = q + lax.dot_general(onehot, lo_ref[i], dims,
                                preferred_element_type=jnp.float32)
        q_st = res + (q - res)
        quant = quant + q_st
        res = res - q_st
        idx_ref[i, :] = idx
    quant_ref[...] = quant


def kernel(x, codebooks):
    b, s, d = x.shape
    num_q, k, _ = codebooks.shape
    tokens = b * s
    x2 = x.reshape(tokens, d)
    cbt = jnp.swapaxes(codebooks, 1, 2)

    mask = jnp.uint32(0xFFFF0000)
    bits = lax.bitcast_convert_type(codebooks, jnp.uint32)
    hi_f = lax.bitcast_convert_type(bits & mask, jnp.float32)
    r1 = codebooks - hi_f
    r1_bits = lax.bitcast_convert_type(r1, jnp.uint32)
    mid_f = lax.bitcast_convert_type(r1_bits & mask, jnp.float32)
    r2 = r1 - mid_f
    cb_hi = hi_f.astype(jnp.bfloat16)
    cb_mid = mid_f.astype(jnp.bfloat16)
    cb_lo = r2.astype(jnp.bfloat16)

    grid = tokens // TB
    quant, idx_all = pl.pallas_call(
        _rvq_body,
        grid=(grid,),
        in_specs=[
            pl.BlockSpec((TB, d), lambda t: (t, 0)),
            pl.BlockSpec((num_q, k, d), lambda t: (0, 0, 0)),
            pl.BlockSpec((num_q, k, d), lambda t: (0, 0, 0)),
            pl.BlockSpec((num_q, k, d), lambda t: (0, 0, 0)),
            pl.BlockSpec((num_q, d, k), lambda t: (0, 0, 0)),
        ],
        out_specs=[
            pl.BlockSpec((TB, d), lambda t: (t, 0)),
            pl.BlockSpec((num_q, TB), lambda t: (0, t)),
        ],
        out_shape=[
            jax.ShapeDtypeStruct((tokens, d), jnp.float32),
            jax.ShapeDtypeStruct((num_q, tokens), jnp.int32),
        ],
    )(x2, cb_hi, cb_mid, cb_lo, cbt)

    quantized = quant.reshape(b, s, d)
    indices = idx_all.T.reshape(b, s, num_q)

    total_commitment_loss = jnp.float32(0.0)
    codebook_usage = jnp.zeros((num_q, k), dtype=jnp.float32)
    usage = codebook_usage / (jnp.sum(codebook_usage, axis=1, keepdims=True) + 1e-05)
    entropy = -jnp.sum(usage * jnp.log(usage + 1e-10), axis=1)
    max_entropy = jnp.log(jnp.float32(k))
    diversity_loss = 1.0 - jnp.mean(entropy) / max_entropy
    total_vq_loss = total_commitment_loss + 0.1 * diversity_loss
    return (quantized, indices, total_commitment_loss, diversity_loss, total_vq_loss)


# speedup vs baseline: 1.0730x; 1.0730x over previous
import jax
import jax.numpy as jnp
from jax import lax
from jax.experimental import pallas as pl

NUM_Q = 8
TB = 512


def _rvq_body(x_ref, hi_ref, mid_ref, lo_ref, cbt_ref, quant_ref, idx_ref):
    res = x_ref[...]
    tb, d = res.shape
    num_q, _, k = cbt_ref.shape
    quant = jnp.zeros_like(res)
    kiota = lax.broadcasted_iota(jnp.int32, (tb, k), 1)
    for i in range(num_q):
        wt = cbt_ref[i]
        scores = lax.dot_general(
            res, wt, (((1,), (0,)), ((), ())),
            preferred_element_type=jnp.float32)
        xsq = jnp.sum(res * res, axis=1, keepdims=True)
        wsq = jnp.sum(wt * wt, axis=0, keepdims=True)
        dist = (xsq + wsq) - 2.0 * scores
        m = jnp.min(dist, axis=1, keepdims=True)
        idx = jnp.min(jnp.where(dist == m, kiota, k), axis=1)
        planes = 3 if i < num_q - 3 else (2 if i < num_q - 1 else 1)
        onehot = (kiota == idx[:, None]).astype(jnp.bfloat16)
        dims = (((1,), (0,)), ((), ()))
        q = lax.dot_general(onehot, hi_ref[i], dims,
                            preferred_element_type=jnp.float32)
        if planes >= 2:
            q = q + lax.dot_general(onehot, mid_ref[i], dims,
                                    preferred_element_type=jnp.float32)
        if planes >= 3:
            q = q + lax.dot_general(onehot, lo_ref[i], dims,
                                    preferred_element_type=jnp.float32)
        q_st = res + (q - res)
        quant = quant + q_st
        res = res - q_st
        idx_ref[i, :] = idx
    quant_ref[...] = quant


def kernel(x, codebooks):
    b, s, d = x.shape
    num_q, k, _ = codebooks.shape
    tokens = b * s
    x2 = x.reshape(tokens, d)
    cbt = jnp.swapaxes(codebooks, 1, 2)

    mask = jnp.uint32(0xFFFF0000)
    bits = lax.bitcast_convert_type(codebooks, jnp.uint32)
    hi_f = lax.bitcast_convert_type(bits & mask, jnp.float32)
    r1 = codebooks - hi_f
    r1_bits = lax.bitcast_convert_type(r1, jnp.uint32)
    mid_f = lax.bitcast_convert_type(r1_bits & mask, jnp.float32)
    r2 = r1 - mid_f
    cb_hi = hi_f.astype(jnp.bfloat16)
    cb_mid = mid_f.astype(jnp.bfloat16)
    cb_lo = r2.astype(jnp.bfloat16)

    grid = tokens // TB
    quant, idx_all = pl.pallas_call(
        _rvq_body,
        grid=(grid,),
        in_specs=[
            pl.BlockSpec((TB, d), lambda t: (t, 0)),
            pl.BlockSpec((num_q, k, d), lambda t: (0, 0, 0)),
            pl.BlockSpec((num_q, k, d), lambda t: (0, 0, 0)),
            pl.BlockSpec((num_q, k, d), lambda t: (0, 0, 0)),
            pl.BlockSpec((num_q, d, k), lambda t: (0, 0, 0)),
        ],
        out_specs=[
            pl.BlockSpec((TB, d), lambda t: (t, 0)),
            pl.BlockSpec((num_q, TB), lambda t: (0, t)),
        ],
        out_shape=[
            jax.ShapeDtypeStruct((tokens, d), jnp.float32),
            jax.ShapeDtypeStruct((num_q, tokens), jnp.int32),
        ],
    )(x2, cb_hi, cb_mid, cb_lo, cbt)

    quantized = quant.reshape(b, s, d)
    indices = idx_all.T.reshape(b, s, num_q)

    total_commitment_loss = jnp.float32(0.0)
    codebook_usage = jnp.zeros((num_q, k), dtype=jnp.float32)
    usage = codebook_usage / (jnp.sum(codebook_usage, axis=1, keepdims=True) + 1e-05)
    entropy = -jnp.sum(usage * jnp.log(usage + 1e-10), axis=1)
    max_entropy = jnp.log(jnp.float32(k))
    diversity_loss = 1.0 - jnp.mean(entropy) / max_entropy
    total_vq_loss = total_commitment_loss + 0.1 * diversity_loss
    return (quantized, indices, total_commitment_loss, diversity_loss, total_vq_loss)


# speedup vs baseline: 1.1482x; 1.0701x over previous
import jax
import jax.numpy as jnp
from jax import lax
from jax.experimental import pallas as pl

NUM_Q = 8
TB = 512


def _rvq_body(x_ref, hi_ref, mid_ref, lo_ref, cbt_ref, quant_ref, idx_ref):
    res = x_ref[...]
    tb, d = res.shape
    num_q, _, k = cbt_ref.shape
    quant = jnp.zeros_like(res)
    kiota = lax.broadcasted_iota(jnp.int32, (tb, k), 1)
    for i in range(num_q):
        wt = cbt_ref[i]
        scores = lax.dot_general(
            res, wt, (((1,), (0,)), ((), ())),
            preferred_element_type=jnp.float32)
        xsq = jnp.sum(res * res, axis=1, keepdims=True)
        wsq = jnp.sum(wt * wt, axis=0, keepdims=True)
        dist = (xsq + wsq) - 2.0 * scores
        m = jnp.min(dist, axis=1, keepdims=True)
        idx = jnp.min(jnp.where(dist == m, kiota, k), axis=1)
        planes = 3 if i < 2 else (2 if i < num_q - 1 else 1)
        onehot = (kiota == idx[:, None]).astype(jnp.bfloat16)
        dims = (((1,), (0,)), ((), ()))
        q = lax.dot_general(onehot, hi_ref[i], dims,
                            preferred_element_type=jnp.float32)
        if planes >= 2:
            q = q + lax.dot_general(onehot, mid_ref[i], dims,
                                    preferred_element_type=jnp.float32)
        if planes >= 3:
            q = q + lax.dot_general(onehot, lo_ref[i], dims,
                                    preferred_element_type=jnp.float32)
        q_st = res + (q - res)
        quant = quant + q_st
        res = res - q_st
        idx_ref[i, :] = idx
    quant_ref[...] = quant


def kernel(x, codebooks):
    b, s, d = x.shape
    num_q, k, _ = codebooks.shape
    tokens = b * s
    x2 = x.reshape(tokens, d)
    cbt = jnp.swapaxes(codebooks, 1, 2)

    mask = jnp.uint32(0xFFFF0000)
    bits = lax.bitcast_convert_type(codebooks, jnp.uint32)
    hi_f = lax.bitcast_convert_type(bits & mask, jnp.float32)
    r1 = codebooks - hi_f
    r1_bits = lax.bitcast_convert_type(r1, jnp.uint32)
    mid_f = lax.bitcast_convert_type(r1_bits & mask, jnp.float32)
    r2 = r1 - mid_f
    cb_hi = hi_f.astype(jnp.bfloat16)
    cb_mid = mid_f.astype(jnp.bfloat16)
    cb_lo = r2.astype(jnp.bfloat16)

    grid = tokens // TB
    quant, idx_all = pl.pallas_call(
        _rvq_body,
        grid=(grid,),
        in_specs=[
            pl.BlockSpec((TB, d), lambda t: (t, 0)),
            pl.BlockSpec((num_q, k, d), lambda t: (0, 0, 0)),
            pl.BlockSpec((num_q, k, d), lambda t: (0, 0, 0)),
            pl.BlockSpec((num_q, k, d), lambda t: (0, 0, 0)),
            pl.BlockSpec((num_q, d, k), lambda t: (0, 0, 0)),
        ],
        out_specs=[
            pl.BlockSpec((TB, d), lambda t: (t, 0)),
            pl.BlockSpec((num_q, TB), lambda t: (0, t)),
        ],
        out_shape=[
            jax.ShapeDtypeStruct((tokens, d), jnp.float32),
            jax.ShapeDtypeStruct((num_q, tokens), jnp.int32),
        ],
    )(x2, cb_hi, cb_mid, cb_lo, cbt)

    quantized = quant.reshape(b, s, d)
    indices = idx_all.T.reshape(b, s, num_q)

    total_commitment_loss = jnp.float32(0.0)
    codebook_usage = jnp.zeros((num_q, k), dtype=jnp.float32)
    usage = codebook_usage / (jnp.sum(codebook_usage, axis=1, keepdims=True) + 1e-05)
    entropy = -jnp.sum(usage * jnp.log(usage + 1e-10), axis=1)
    max_entropy = jnp.log(jnp.float32(k))
    diversity_loss = 1.0 - jnp.mean(entropy) / max_entropy
    total_vq_loss = total_commitment_loss + 0.1 * diversity_loss
    return (quantized, indices, total_commitment_loss, diversity_loss, total_vq_loss)
